# Initial kernel scaffold; baseline (speedup 1.0000x reference)
import jax
import jax.numpy as jnp
from jax.experimental import pallas as pl


def kernel(x, table, W, b):
    raise NotImplementedError("write your pallas kernel here")


# speedup vs baseline: 1.9895x; 1.9895x over previous
import functools

import jax
import jax.numpy as jnp
from jax import lax
from jax.experimental import pallas as pl
from jax.experimental.pallas import tpu as pltpu
from jax.experimental.pallas import tpu_sc as plsc


def _matmul_body(t_ref, wt_ref, p_ref):
    p_ref[...] = jnp.dot(t_ref[...], wt_ref[...],
                         preferred_element_type=jnp.float32)


def _project_table(table, Wt, rows_per_block):
    vocab, emb = table.shape
    ncls = Wt.shape[1]
    grid = vocab // rows_per_block
    return pl.pallas_call(
        _matmul_body,
        grid=(grid,),
        in_specs=[
            pl.BlockSpec((rows_per_block, emb), lambda i: (i, 0)),
            pl.BlockSpec((emb, ncls), lambda i: (0, 0)),
        ],
        out_specs=pl.BlockSpec((rows_per_block, ncls), lambda i: (i, 0)),
        out_shape=jax.ShapeDtypeStruct((vocab, ncls), jnp.float32),
    )(table, Wt)


def _make_sc_pool(B, LP, vocab):
    info = plsc.get_sparse_core_info()
    NW = info.num_cores * info.num_subcores
    SPT = B // NW
    CHUNK = 32
    NCHUNK = SPT // CHUNK
    TOK_CHUNK = CHUNK * LP
    GROUPS = CHUNK // 8
    DP = 8

    mesh = plsc.VectorSubcoreMesh(core_axis_name="c", subcore_axis_name="s")

    @functools.partial(
        pl.kernel,
        out_type=jax.ShapeDtypeStruct((B * 2,), jnp.float32),
        mesh=mesh,
        compiler_params=pltpu.CompilerParams(needs_layout_passes=False,
                                             use_tc_tiling_on_sc=False),
        scratch_types=[
            pltpu.VMEM((TOK_CHUNK,), jnp.int32),
            pltpu.VMEM((TOK_CHUNK, DP), jnp.float32),
            pltpu.VMEM((SPT * 2,), jnp.float32),
            pltpu.VMEM((16,), jnp.float32),
            pltpu.SemaphoreType.DMA,
        ],
    )
    def sc_pool(p_hbm, xflat_hbm, btile_hbm, out_hbm,
                idx_v, rows_v, out_v, b_v, sem):
        cid = lax.axis_index("c")
        sid = lax.axis_index("s")
        wid = sid * info.num_cores + cid
        tile_tok_base = wid * (SPT * LP)

        pltpu.sync_copy(btile_hbm, b_v)
        bvec = b_v[...]

        lane = lax.iota(jnp.int32, 16)
        m = lane >> 1
        cl = lane & 1

        zf = jnp.zeros((16,), jnp.float32)
        onef = jnp.full((16,), 1.0, jnp.float32)

        for chunk in range(NCHUNK):
            tok0 = tile_tok_base + chunk * TOK_CHUNK
            pltpu.sync_copy(xflat_hbm.at[pl.ds(tok0, TOK_CHUNK)], idx_v)
            pltpu.async_copy(p_hbm.at[idx_v], rows_v, sem).wait()

            for g in range(GROUPS):
                ri0 = (g * 8 * LP) + m * LP

                def step(t, carry):
                    acc, cnt = carry
                    vals = plsc.load_gather(rows_v, [ri0 + t, cl])
                    iv = plsc.load_gather(idx_v, [ri0 + t])
                    acc = acc + vals
                    cnt = cnt + jnp.where(iv != 0, onef, zf)
                    return (acc, cnt)

                acc, cnt = lax.fori_loop(0, LP, step, (zf, zf))
                cnt = jnp.maximum(cnt, 1.0)
                res = acc / cnt + bvec
                out_v[pl.ds((chunk * CHUNK + g * 8) * 2, 16)] = res

        pltpu.sync_copy(out_v, out_hbm.at[pl.ds(wid * SPT * 2, SPT * 2)])

    return sc_pool


def kernel(x, table, W, b):
    B, L = x.shape
    vocab, emb = table.shape
    ncls = W.shape[0]

    LP = ((L + 15) // 16) * 16
    xpad = jnp.pad(x.astype(jnp.int32), ((0, 0), (0, LP - L)))
    xflat = xpad.reshape(B * LP)

    Wt_pad = jnp.pad(W.T, ((0, 0), (0, 8 - ncls)))
    P = _project_table(table, Wt_pad, rows_per_block=10000)

    btile = jnp.tile(b, 16 // ncls)
    out_flat = _make_sc_pool(B, LP, vocab)(P, xflat, btile)
    return out_flat.reshape(B, ncls)


# speedup vs baseline: 1.9981x; 1.0043x over previous
import functools

import jax
import jax.numpy as jnp
from jax import lax
from jax.experimental import pallas as pl
from jax.experimental.pallas import tpu as pltpu
from jax.experimental.pallas import tpu_sc as plsc

_DP = 8
_CNT_COL = 2


def _matmul_body(t_ref, wt_ref, p_ref):
    res = jnp.dot(t_ref[...], wt_ref[...],
                  preferred_element_type=jnp.float32)
    col = lax.broadcasted_iota(jnp.int32, res.shape, 1)
    row = lax.broadcasted_iota(jnp.int32, res.shape, 0)
    first = pl.program_id(0) == 0
    is_pad_row = jnp.logical_and(first, row == 0)
    sentinel = jnp.where(is_pad_row, 0.0, 1.0)
    p_ref[...] = jnp.where(col == _CNT_COL, sentinel, res)


def _project_table(table, Wt, rows_per_block):
    vocab, emb = table.shape
    grid = vocab // rows_per_block
    return pl.pallas_call(
        _matmul_body,
        grid=(grid,),
        in_specs=[
            pl.BlockSpec((rows_per_block, emb), lambda i: (i, 0)),
            pl.BlockSpec((emb, _DP), lambda i: (0, 0)),
        ],
        out_specs=pl.BlockSpec((rows_per_block, _DP), lambda i: (i, 0)),
        out_shape=jax.ShapeDtypeStruct((vocab, _DP), jnp.float32),
    )(table, Wt)


def _make_sc_pool(B, LP, vocab):
    info = plsc.get_sparse_core_info()
    NW = info.num_cores * info.num_subcores
    SPT = B // NW
    CHUNK = 32
    NCHUNK = SPT // CHUNK
    TOK_CHUNK = CHUNK * LP
    GROUPS = CHUNK // 8

    mesh = plsc.VectorSubcoreMesh(core_axis_name="c", subcore_axis_name="s")

    @functools.partial(
        pl.kernel,
        out_type=jax.ShapeDtypeStruct((B * 2,), jnp.float32),
        mesh=mesh,
        compiler_params=pltpu.CompilerParams(needs_layout_passes=False,
                                             use_tc_tiling_on_sc=False),
        scratch_types=[
            pltpu.VMEM((2, TOK_CHUNK), jnp.int32),
            pltpu.VMEM((2 * TOK_CHUNK, _DP), jnp.float32),
            pltpu.VMEM((SPT * 2,), jnp.float32),
            pltpu.VMEM((16,), jnp.float32),
            pltpu.SemaphoreType.DMA,
            pltpu.SemaphoreType.DMA,
        ],
    )
    def sc_pool(p_hbm, xflat_hbm, btile_hbm, out_hbm,
                idx_v, rows_v, out_v, b_v, sem0, sem1):
        cid = lax.axis_index("c")
        sid = lax.axis_index("s")
        wid = sid * info.num_cores + cid
        tile_tok_base = wid * (SPT * LP)
        sems = (sem0, sem1)

        pltpu.sync_copy(btile_hbm, b_v)
        bvec = b_v[...]

        lane = lax.iota(jnp.int32, 16)
        m = lane >> 1
        cl = lane & 1
        ccol = jnp.full((16,), _CNT_COL, jnp.int32)

        zf = jnp.zeros((16,), jnp.float32)

        def stage(chunk, buf):
            tok0 = tile_tok_base + chunk * TOK_CHUNK
            pltpu.sync_copy(xflat_hbm.at[pl.ds(tok0, TOK_CHUNK)],
                            idx_v.at[buf])
            return pltpu.async_copy(
                p_hbm.at[idx_v.at[buf]],
                rows_v.at[pl.ds(buf * TOK_CHUNK, TOK_CHUNK)], sems[buf])

        stage(0, 0)
        for chunk in range(NCHUNK):
            cur = chunk & 1
            if chunk + 1 < NCHUNK:
                stage(chunk + 1, 1 - cur)
            pltpu.make_async_copy(
                p_hbm.at[idx_v.at[cur]],
                rows_v.at[pl.ds(cur * TOK_CHUNK, TOK_CHUNK)],
                sems[cur]).wait()

            rbase = cur * TOK_CHUNK
            for g in range(GROUPS):
                ri0 = rbase + (g * 8 * LP) + m * LP

                def step(t, carry):
                    acc, cnt = carry
                    ri = ri0 + t
                    vals = plsc.load_gather(rows_v, [ri, cl])
                    cvals = plsc.load_gather(rows_v, [ri, ccol])
                    return (acc + vals, cnt + cvals)

                acc, cnt = lax.fori_loop(0, LP, step, (zf, zf), unroll=8)
                cnt = jnp.maximum(cnt, 1.0)
                res = acc / cnt + bvec
                out_v[pl.ds((chunk * CHUNK + g * 8) * 2, 16)] = res

        pltpu.sync_copy(out_v, out_hbm.at[pl.ds(wid * SPT * 2, SPT * 2)])

    return sc_pool


def kernel(x, table, W, b):
    B, L = x.shape
    vocab, emb = table.shape
    ncls = W.shape[0]

    LP = ((L + 15) // 16) * 16
    xpad = jnp.pad(x.astype(jnp.int32), ((0, 0), (0, LP - L)))
    xflat = xpad.reshape(B * LP)

    Wt_pad = jnp.pad(W.T, ((0, 0), (0, _DP - ncls)))
    P = _project_table(table, Wt_pad, rows_per_block=10000)

    btile = jnp.tile(b, 16 // ncls)
    out_flat = _make_sc_pool(B, LP, vocab)(P, xflat, btile)
    return out_flat.reshape(B, ncls)


# speedup vs baseline: 2.2381x; 1.1201x over previous
import functools

import jax
import jax.numpy as jnp
from jax import lax
from jax.experimental import pallas as pl
from jax.experimental.pallas import tpu as pltpu
from jax.experimental.pallas import tpu_sc as plsc

_DP = 16
_CNT_COL = 2


def _matmul_body(t2_ref, m_ref, p_ref):
    res = jnp.dot(t2_ref[...], m_ref[...],
                  preferred_element_type=jnp.float32)
    col = lax.broadcasted_iota(jnp.int32, res.shape, 1)
    row = lax.broadcasted_iota(jnp.int32, res.shape, 0)
    first = pl.program_id(0) == 0
    is_pad = jnp.logical_and(first, jnp.logical_and(row == 0,
                                                    col == _CNT_COL))
    is_cnt = (col & (_DP - 1)) == _CNT_COL
    res = jnp.where(is_cnt, jnp.where(is_pad, 0.0, 1.0), res)
    p_ref[...] = res


def _project_table(t2, M, rows_per_block):
    n_rows, k = t2.shape
    grid = n_rows // rows_per_block
    return pl.pallas_call(
        _matmul_body,
        grid=(grid,),
        in_specs=[
            pl.BlockSpec((rows_per_block, k), lambda i: (i, 0)),
            pl.BlockSpec((k, 128), lambda i: (0, 0)),
        ],
        out_specs=pl.BlockSpec((rows_per_block, 128), lambda i: (i, 0)),
        out_shape=jax.ShapeDtypeStruct((n_rows, 128), jnp.float32),
    )(t2, M)


def _make_sc_pool(B, LP, vocab):
    info = plsc.get_sparse_core_info()
    NW = info.num_cores * info.num_subcores
    SPT = B // NW
    CHUNK = 16
    NCHUNK = SPT // CHUNK
    TOK_CHUNK = CHUNK * LP
    GROUPS = CHUNK // 8

    mesh = plsc.VectorSubcoreMesh(core_axis_name="c", subcore_axis_name="s")

    @functools.partial(
        pl.kernel,
        out_type=jax.ShapeDtypeStruct((B * 2,), jnp.float32),
        mesh=mesh,
        compiler_params=pltpu.CompilerParams(needs_layout_passes=False,
                                             use_tc_tiling_on_sc=False),
        scratch_types=[
            pltpu.VMEM((2, TOK_CHUNK), jnp.int32),
            pltpu.VMEM((2 * TOK_CHUNK, _DP), jnp.float32),
            pltpu.VMEM((SPT * 2,), jnp.float32),
            pltpu.VMEM((16,), jnp.float32),
            pltpu.SemaphoreType.DMA,
            pltpu.SemaphoreType.DMA,
        ],
    )
    def sc_pool(p_hbm, xflat_hbm, btile_hbm, out_hbm,
                idx_v, rows_v, out_v, b_v, sem0, sem1):
        cid = lax.axis_index("c")
        sid = lax.axis_index("s")
        wid = sid * info.num_cores + cid
        tile_tok_base = wid * (SPT * LP)
        sems = (sem0, sem1)

        pltpu.sync_copy(btile_hbm, b_v)
        bvec = b_v[...]

        lane = lax.iota(jnp.int32, 16)
        m = lane >> 1
        cl = lane & 1
        ccol = jnp.full((16,), _CNT_COL, jnp.int32)

        zf = jnp.zeros((16,), jnp.float32)

        def stage(chunk, buf):
            tok0 = tile_tok_base + chunk * TOK_CHUNK
            pltpu.sync_copy(xflat_hbm.at[pl.ds(tok0, TOK_CHUNK)],
                            idx_v.at[buf])
            return pltpu.async_copy(
                p_hbm.at[idx_v.at[buf]],
                rows_v.at[pl.ds(buf * TOK_CHUNK, TOK_CHUNK)], sems[buf])

        stage(0, 0)
        for chunk in range(NCHUNK):
            cur = chunk & 1
            if chunk + 1 < NCHUNK:
                stage(chunk + 1, 1 - cur)
            pltpu.make_async_copy(
                p_hbm.at[idx_v.at[cur]],
                rows_v.at[pl.ds(cur * TOK_CHUNK, TOK_CHUNK)],
                sems[cur]).wait()

            rbase = cur * TOK_CHUNK
            for g in range(GROUPS):
                ri0 = rbase + (g * 8 * LP) + m * LP

                def step(t, carry):
                    acc, cnt = carry
                    ri = ri0 + t
                    vals = plsc.load_gather(rows_v, [ri, cl])
                    cvals = plsc.load_gather(rows_v, [ri, ccol])
                    return (acc + vals, cnt + cvals)

                acc, cnt = lax.fori_loop(0, LP, step, (zf, zf), unroll=8)
                cnt = jnp.maximum(cnt, 1.0)
                res = acc / cnt + bvec
                out_v[pl.ds((chunk * CHUNK + g * 8) * 2, 16)] = res

        pltpu.sync_copy(out_v, out_hbm.at[pl.ds(wid * SPT * 2, SPT * 2)])

    return sc_pool


def kernel(x, table, W, b):
    B, L = x.shape
    vocab, emb = table.shape
    ncls = W.shape[0]

    LP = ((L + 15) // 16) * 16
    xpad = jnp.pad(x.astype(jnp.int32), ((0, 0), (0, LP - L)))
    xflat = xpad.reshape(B * LP)

    Wt_pad = jnp.pad(W.T, ((0, 0), (0, _DP - ncls)))
    vpg = 128 // _DP
    M = jnp.kron(jnp.eye(vpg, dtype=jnp.float32), Wt_pad)
    t2 = table.reshape(vocab // vpg, vpg * emb)
    P_wide = _project_table(t2, M, rows_per_block=1000)
    P = P_wide.reshape(vocab, _DP)

    btile = jnp.tile(b, 16 // ncls)
    out_flat = _make_sc_pool(B, LP, vocab)(P, xflat, btile)
    return out_flat.reshape(B, ncls)
